# Initial kernel scaffold; baseline (speedup 1.0000x reference)
import jax
import jax.numpy as jnp
from jax.experimental import pallas as pl


def kernel(h, edge_follows, edge_likes, W0_f, b0_f, W0_l, b0_l, W1_f, b1_f, W1_l, b1_l):
    raise NotImplementedError("write your pallas kernel here")


# speedup vs baseline: 5.7837x; 5.7837x over previous
import functools

import jax
import jax.numpy as jnp
from jax import lax
from jax.experimental import pallas as pl
from jax.experimental.pallas import tpu as pltpu
from jax.experimental.pallas import tpu_sc as plsc

N = 10000
D = 128
E = 320000
NSUB = 16
M = 10112
TPT = E // NSUB
CH = 128
KCH = M // CH
NCH = 158
PT = NCH * CH
BR = 1264
NBLK = M // BR

_sc_mesh = plsc.VectorSubcoreMesh(core_axis_name="c", subcore_axis_name="s")


def _each_chunk(s, fn):
    for j in range(4):
        fn(s + 16 * j)
    q4 = s + 64

    @pl.when(q4 < KCH)
    def _():
        fn(q4)


@functools.partial(
    pl.kernel,
    out_type=jax.ShapeDtypeStruct((4 * M,), jnp.float32),
    mesh=_sc_mesh,
    scratch_types=[
        pltpu.VMEM((CH,), jnp.float32),
        pltpu.VMEM((CH,), jnp.float32),
        pltpu.VMEM((CH,), jnp.int32),
        pltpu.VMEM((CH,), jnp.int32),
        pltpu.VMEM_SHARED((M,), jnp.float32),
        pltpu.VMEM_SHARED((M,), jnp.float32),
    ],
)
def _deg_kernel(dsrc, ddst, zvec, ones, deg_out,
                zv, ov, sidx, didx, out_acc, in_acc):
    c = lax.axis_index("c")
    s = lax.axis_index("s")
    pltpu.sync_copy(zvec, zv)
    pltpu.sync_copy(ones, ov)

    def _zero(q):
        pltpu.sync_copy(zv, out_acc.at[pl.ds(q * CH, CH)])
        pltpu.sync_copy(zv, in_acc.at[pl.ds(q * CH, CH)])

    _each_chunk(s, _zero)
    plsc.subcore_barrier()
    base = (c * NSUB + s) * PT

    def body(g, carry):
        off = base + g * CH
        pltpu.sync_copy(dsrc.at[pl.ds(off, CH)], sidx)
        pltpu.sync_copy(ddst.at[pl.ds(off, CH)], didx)
        pltpu.sync_copy(ov, out_acc.at[sidx], add=True)
        pltpu.sync_copy(ov, in_acc.at[didx], add=True)
        return carry

    lax.fori_loop(0, NCH, body, 0)
    plsc.subcore_barrier()

    def _wb(q):
        pltpu.sync_copy(out_acc.at[pl.ds(q * CH, CH)], zv)
        pltpu.sync_copy(zv, deg_out.at[pl.ds((2 * c) * M + q * CH, CH)])
        pltpu.sync_copy(in_acc.at[pl.ds(q * CH, CH)], zv)
        pltpu.sync_copy(zv, deg_out.at[pl.ds((2 * c + 1) * M + q * CH, CH)])

    _each_chunk(s, _wb)


@functools.partial(
    pl.kernel,
    out_type=jax.ShapeDtypeStruct((2, M, D), jnp.float32),
    mesh=_sc_mesh,
    scratch_types=[
        pltpu.VMEM((CH, D), jnp.float32),
        pltpu.VMEM((CH, D), jnp.float32),
        pltpu.VMEM((CH, D), jnp.float32),
        pltpu.VMEM((CH,), jnp.int32),
        pltpu.VMEM((CH,), jnp.int32),
        pltpu.VMEM((CH,), jnp.int32),
        pltpu.VMEM((CH,), jnp.int32),
        pltpu.VMEM_SHARED((M, D), jnp.float32),
        pltpu.SemaphoreType.DMA,
        pltpu.SemaphoreType.DMA,
    ],
)
def _agg_kernel(table, gsrc, gdst, zblk, agg_out,
                zb, rows0, rows1, sidx0, didx0, sidx1, didx1,
                acc, sem0, sem1):
    c = lax.axis_index("c")
    s = lax.axis_index("s")
    pltpu.sync_copy(zblk, zb)
    _each_chunk(s, lambda q: pltpu.sync_copy(zb, acc.at[pl.ds(q * CH, CH)]))
    plsc.subcore_barrier()
    base = (c * NSUB + s) * PT

    pltpu.sync_copy(gsrc.at[pl.ds(base, CH)], sidx0)
    pltpu.sync_copy(gdst.at[pl.ds(base, CH)], didx0)
    pltpu.async_copy(table.at[sidx0], rows0, sem0)

    def body(i, carry):
        offb = base + (2 * i + 1) * CH
        pltpu.sync_copy(gsrc.at[pl.ds(offb, CH)], sidx1)
        pltpu.sync_copy(gdst.at[pl.ds(offb, CH)], didx1)
        cp1 = pltpu.async_copy(table.at[sidx1], rows1, sem1)
        pltpu.make_async_copy(table.at[sidx0], rows0, sem0).wait()
        pltpu.sync_copy(rows0, acc.at[didx0], add=True)

        @pl.when(2 * i + 2 < NCH)
        def _prefetch():
            offn = base + (2 * i + 2) * CH
            pltpu.sync_copy(gsrc.at[pl.ds(offn, CH)], sidx0)
            pltpu.sync_copy(gdst.at[pl.ds(offn, CH)], didx0)
            pltpu.async_copy(table.at[sidx0], rows0, sem0)

        cp1.wait()
        pltpu.sync_copy(rows1, acc.at[didx1], add=True)
        return carry

    lax.fori_loop(0, NCH // 2, body, 0)
    plsc.subcore_barrier()

    def _wb(q):
        pltpu.sync_copy(acc.at[pl.ds(q * CH, CH)], zb)
        pltpu.sync_copy(zb, agg_out.at[c, pl.ds(q * CH, CH)])

    _each_chunk(s, _wb)


def _rs(x):
    return lax.rsqrt(jnp.maximum(x, 1.0))


def _mm0_body(h_ref, w_ref, deg_ref, o_ref):
    r = pl.program_id(0)
    d = deg_ref[...]
    sc = _rs(jnp.where(r == 0, d[:, 0], d[:, 2]))
    o_ref[0] = jnp.dot(h_ref[...], w_ref[0],
                       preferred_element_type=jnp.float32) * sc[:, None]


_mm0 = pl.pallas_call(
    _mm0_body,
    grid=(2, NBLK),
    in_specs=[
        pl.BlockSpec((BR, D), lambda r, i: (i, 0)),
        pl.BlockSpec((1, D, D), lambda r, i: (r, 0, 0)),
        pl.BlockSpec((BR, 4), lambda r, i: (i, 0)),
    ],
    out_specs=pl.BlockSpec((1, BR, D), lambda r, i: (r, i, 0)),
    out_shape=jax.ShapeDtypeStruct((2, M, D), jnp.float32),
)


def _mm1_body(af_ref, al_ref, deg_ref, b0_ref, w_ref, o_ref):
    r = pl.program_id(0)
    d = deg_ref[...]
    h0 = (af_ref[0] * _rs(d[:, 1])[:, None] + b0_ref[0][None, :]
          + al_ref[0] * _rs(d[:, 3])[:, None] + b0_ref[1][None, :])
    h0 = jnp.maximum(h0, 0.0)
    sc = _rs(jnp.where(r == 0, d[:, 0], d[:, 2]))
    o_ref[0] = jnp.dot(h0, w_ref[0],
                       preferred_element_type=jnp.float32) * sc[:, None]


_mm1 = pl.pallas_call(
    _mm1_body,
    grid=(2, NBLK),
    in_specs=[
        pl.BlockSpec((1, BR, D), lambda r, i: (0, i, 0)),
        pl.BlockSpec((1, BR, D), lambda r, i: (1, i, 0)),
        pl.BlockSpec((BR, 4), lambda r, i: (i, 0)),
        pl.BlockSpec((2, D), lambda r, i: (0, 0)),
        pl.BlockSpec((1, D, D), lambda r, i: (r, 0, 0)),
    ],
    out_specs=pl.BlockSpec((1, BR, D), lambda r, i: (r, i, 0)),
    out_shape=jax.ShapeDtypeStruct((2, M, D), jnp.float32),
)


def _fin_body(af_ref, al_ref, deg_ref, b1_ref, o_ref):
    d = deg_ref[...]
    o_ref[...] = (af_ref[0] * _rs(d[:, 1])[:, None] + b1_ref[0][None, :]
                  + al_ref[0] * _rs(d[:, 3])[:, None] + b1_ref[1][None, :])


_fin = pl.pallas_call(
    _fin_body,
    grid=(NBLK,),
    in_specs=[
        pl.BlockSpec((1, BR, D), lambda i: (0, i, 0)),
        pl.BlockSpec((1, BR, D), lambda i: (1, i, 0)),
        pl.BlockSpec((BR, 4), lambda i: (i, 0)),
        pl.BlockSpec((2, D), lambda i: (0, 0)),
    ],
    out_specs=pl.BlockSpec((BR, D), lambda i: (i, 0)),
    out_shape=jax.ShapeDtypeStruct((M, D), jnp.float32),
)


def _pad_tiles(x, padval):
    x = x.reshape(NSUB, TPT)
    pad = jnp.full((NSUB, PT - TPT), padval, jnp.int32)
    return jnp.concatenate([x, pad], axis=1).reshape(-1)


def kernel(h, edge_follows, edge_likes,
           W0_f, b0_f, W0_l, b0_l, W1_f, b1_f, W1_l, b1_l):
    h_pad = jnp.zeros((M, D), jnp.float32).at[:N].set(h)
    sf, df = edge_follows[0], edge_follows[1]
    sl, dl = edge_likes[0], edge_likes[1]
    dsrc = jnp.concatenate([_pad_tiles(sf, N), _pad_tiles(sl, N)])
    gsrc = jnp.concatenate([_pad_tiles(sf, 0), _pad_tiles(sl + M, M)])
    gdst = jnp.concatenate([_pad_tiles(df, N), _pad_tiles(dl, N)])
    zvec = jnp.zeros((CH,), jnp.float32)
    ones = jnp.ones((CH,), jnp.float32)
    zblk = jnp.zeros((CH, D), jnp.float32)

    deg = _deg_kernel(dsrc, gdst, zvec, ones).reshape(4, M).T

    w0 = jnp.stack([W0_f, W0_l])
    w1 = jnp.stack([W1_f, W1_l])
    b0 = jnp.stack([b0_f, b0_l])
    b1 = jnp.stack([b1_f, b1_l])

    hw0 = _mm0(h_pad, w0, deg)
    agg0 = _agg_kernel(hw0.reshape(2 * M, D), gsrc, gdst, zblk)
    hw1 = _mm1(agg0, agg0, deg, b0, w1)
    agg1 = _agg_kernel(hw1.reshape(2 * M, D), gsrc, gdst, zblk)
    out_full = _fin(agg1, agg1, deg, b1)
    return out_full[:N]
